# Initial kernel scaffold; baseline (speedup 1.0000x reference)
import functools

import jax
import jax.numpy as jnp
from jax import lax
from jax.experimental import pallas as pl
from jax.experimental.pallas import tpu as pltpu
from jax.experimental.pallas import tpu_sc as plsc

NUM_EMB = 1000000
DIM = 32
BATCH = 16384
SEQ = 20
TOTAL = BATCH * SEQ

NW = 32
PER_W = TOTAL // NW
CHUNK = 1024
NCHUNK = PER_W // CHUNK


def _emb_body(ids_hbm, tab_hbm, out_hbm, idx_v, rows_v, sem):
    c = lax.axis_index("c")
    s = lax.axis_index("s")
    wid = s * 2 + c
    base = wid * PER_W

    def body(i, carry):
        off = base + i * CHUNK
        pltpu.sync_copy(ids_hbm.at[pl.ds(off, CHUNK)], idx_v)
        pltpu.async_copy(tab_hbm.at[idx_v], rows_v, sem).wait()
        pltpu.sync_copy(rows_v, out_hbm.at[pl.ds(off, CHUNK)])
        return carry

    lax.fori_loop(0, NCHUNK, body, 0)


@jax.jit
def _emb(ids, weight):
    mesh = plsc.VectorSubcoreMesh(core_axis_name="c", subcore_axis_name="s")
    f = functools.partial(
        pl.kernel,
        mesh=mesh,
        out_type=jax.ShapeDtypeStruct((TOTAL, DIM), jnp.float32),
        scratch_types=[
            pltpu.VMEM((CHUNK,), jnp.int32),
            pltpu.VMEM((CHUNK, DIM), jnp.float32),
            pltpu.SemaphoreType.DMA,
        ],
    )(_emb_body)
    return f(ids, weight)


def kernel(token_ids, weight):
    ids = token_ids.reshape(-1).astype(jnp.int32)
    out = _emb(ids, weight)
    return out.reshape(BATCH, SEQ, DIM)


# speedup vs baseline: 1.4922x; 1.4922x over previous
import functools

import jax
import jax.numpy as jnp
from jax import lax
from jax.experimental import pallas as pl
from jax.experimental.pallas import tpu as pltpu
from jax.experimental.pallas import tpu_sc as plsc

NUM_EMB = 1000000
DIM = 32
BATCH = 16384
SEQ = 20
TOTAL = BATCH * SEQ

NW = 32
PER_W = TOTAL // NW
CHUNK = 1024
NCHUNK = PER_W // CHUNK


def _emb_body(ids_hbm, tab_hbm, out_hbm, idx_v, rows_v, sem):
    c = lax.axis_index("c")
    s = lax.axis_index("s")
    wid = s * 2 + c
    base = wid * PER_W

    def body(i, carry):
        off = base + i * CHUNK
        pltpu.sync_copy(ids_hbm.at[pl.ds(off, CHUNK)], idx_v)
        pltpu.async_copy(tab_hbm.at[idx_v], rows_v, sem).wait()
        pltpu.sync_copy(rows_v, out_hbm.at[pl.ds(off, CHUNK)])
        return carry

    lax.fori_loop(0, NCHUNK, body, 0)


@jax.jit
def _emb(ids, weight):
    mesh = plsc.VectorSubcoreMesh(core_axis_name="c", subcore_axis_name="s")
    f = functools.partial(
        pl.kernel,
        mesh=mesh,
        out_type=jax.ShapeDtypeStruct((TOTAL, DIM), jnp.float32),
        scratch_types=[
            pltpu.VMEM((CHUNK,), jnp.int32),
            pltpu.VMEM((CHUNK, DIM), jnp.float32),
            pltpu.SemaphoreType.DMA,
        ],
        compiler_params=pltpu.CompilerParams(use_tc_tiling_on_sc=False),
    )(_emb_body)
    return f(ids, weight)


def kernel(token_ids, weight):
    ids = token_ids.reshape(-1).astype(jnp.int32)
    out = _emb(ids, weight)
    return out.reshape(BATCH, SEQ, DIM)


# speedup vs baseline: 1.5131x; 1.0140x over previous
import functools

import jax
import jax.numpy as jnp
from jax import lax
from jax.experimental import pallas as pl
from jax.experimental.pallas import tpu as pltpu
from jax.experimental.pallas import tpu_sc as plsc

NUM_EMB = 1000000
DIM = 32
BATCH = 16384
SEQ = 20
TOTAL = BATCH * SEQ

NW = 32
PER_W = TOTAL // NW
CHUNK = 1024
NCHUNK = PER_W // CHUNK
NBUF = 2


def _emb_body(ids_hbm, tab_hbm, out_hbm, idx_v, rows_v,
              isem, gsem0, gsem1, ssem0, ssem1):
    c = lax.axis_index("c")
    s = lax.axis_index("s")
    wid = s * 2 + c
    base = wid * PER_W
    gsems = [gsem0, gsem1]
    ssems = [ssem0, ssem1]

    idx_copies = [
        pltpu.async_copy(ids_hbm.at[pl.ds(base + i * CHUNK, CHUNK)],
                         idx_v.at[i], isem)
        for i in range(NCHUNK)
    ]
    for cp in idx_copies:
        cp.wait()

    gathers = [None] * NBUF
    stores = [None] * NBUF
    for i in range(NCHUNK + 1):
        if i < NCHUNK:
            b = i % NBUF
            if stores[b] is not None:
                stores[b].wait()
                stores[b] = None
            gathers[b] = pltpu.async_copy(
                tab_hbm.at[idx_v.at[i]], rows_v.at[b], gsems[b])
        j = i - 1
        if j >= 0:
            bj = j % NBUF
            gathers[bj].wait()
            stores[bj] = pltpu.async_copy(
                rows_v.at[bj], out_hbm.at[pl.ds(base + j * CHUNK, CHUNK)],
                ssems[bj])
    for st in stores:
        if st is not None:
            st.wait()


@jax.jit
def _emb(ids, weight):
    mesh = plsc.VectorSubcoreMesh(core_axis_name="c", subcore_axis_name="s")
    f = functools.partial(
        pl.kernel,
        mesh=mesh,
        out_type=jax.ShapeDtypeStruct((TOTAL, DIM), jnp.float32),
        scratch_types=[
            pltpu.VMEM((NCHUNK, CHUNK), jnp.int32),
            pltpu.VMEM((NBUF, CHUNK, DIM), jnp.float32),
            pltpu.SemaphoreType.DMA,
            pltpu.SemaphoreType.DMA,
            pltpu.SemaphoreType.DMA,
            pltpu.SemaphoreType.DMA,
            pltpu.SemaphoreType.DMA,
        ],
        compiler_params=pltpu.CompilerParams(use_tc_tiling_on_sc=False),
    )(_emb_body)
    return f(ids, weight)


def kernel(token_ids, weight):
    ids = token_ids.reshape(-1).astype(jnp.int32)
    out = _emb(ids, weight)
    return out.reshape(BATCH, SEQ, DIM)
